# baseline (speedup 1.0000x reference)
import jax
import jax.numpy as jnp
from jax import lax
from jax.experimental import pallas as pl
from jax.experimental.pallas import tpu as pltpu
from jax.experimental.pallas import tpu_sc as plsc

_INFO = plsc.get_sparse_core_info()
_NC = _INFO.num_cores
_NS = _INFO.num_subcores
_NW = _NC * _NS

CHARSET = 1000
_WPAD = 1024
B, L = 4096, 20
_N = B * L
_PER_W = _N // _NW
_C = 20
_NCHUNK = _PER_W // _C
_NBUF = 4
_T = _NCHUNK // _NBUF
_REM = _NCHUNK - (_T - 1) * _NBUF

_EB = 16


def _gather_body(table_hbm, idx_hbm, out_hbm, idx_v,
                 r0, r1, r2, r3, g0, g1, g2, g3, w0, w1, w2, w3):
    bufs = [r0, r1, r2, r3]
    gs = [g0, g1, g2, g3]
    ws = [w0, w1, w2, w3]
    wid = lax.axis_index("s") * _NC + lax.axis_index("c")
    base = wid * _PER_W
    pltpu.sync_copy(idx_hbm.at[wid], idx_v)

    def start_gather(j, b):
        pltpu.async_copy(table_hbm.at[idx_v.at[j]], bufs[b], gs[b])

    def wait_gather(j, b):
        pltpu.make_async_copy(table_hbm.at[idx_v.at[j]], bufs[b], gs[b]).wait()

    def start_write(j, b):
        pltpu.async_copy(
            bufs[b], out_hbm.at[pl.ds(base + j * _C, _C)], ws[b])

    def wait_write(j, b):
        pltpu.make_async_copy(
            bufs[b], out_hbm.at[pl.ds(base + j * _C, _C)], ws[b]).wait()

    for b in range(_NBUF):
        start_gather(b, b)

    def outer(g, carry):
        jj = g * _NBUF
        for b in range(_NBUF):
            wait_gather(jj + b, b)
            start_write(jj + b, b)
        for b in range(_NBUF):
            wait_write(jj + b, b)
            start_gather(jj + _NBUF + b, b)
        return carry

    lax.fori_loop(0, _T - 1, outer, 0)

    jj = (_T - 1) * _NBUF
    for k in range(_REM):
        b = k % _NBUF
        wait_gather(jj + k, b)
        start_write(jj + k, b)
    for k in range(_REM):
        b = k % _NBUF
        wait_write(jj + k, b)


def _compact_body(x_ref, y_ref):
    x = x_ref[...].reshape(_EB, L, _WPAD)
    y_ref[...] = x[:, :, :CHARSET]


@jax.jit
def _run(table3, idx3):
    mesh = plsc.VectorSubcoreMesh(core_axis_name="c", subcore_axis_name="s")
    gather = pl.kernel(
        _gather_body,
        out_type=jax.ShapeDtypeStruct((_N, 8, 128), jnp.float32),
        mesh=mesh,
        scratch_types=(
            [pltpu.VMEM((_NCHUNK, _C), jnp.int32)]
            + [pltpu.VMEM((_C, 8, 128), jnp.float32) for _ in range(_NBUF)]
            + [pltpu.SemaphoreType.DMA for _ in range(2 * _NBUF)]
        ),
    )
    rows_p = gather(table3, idx3)
    compact = pl.pallas_call(
        _compact_body,
        grid=(B // _EB,),
        in_specs=[pl.BlockSpec((_EB * L, 8, 128), lambda i: (i, 0, 0))],
        out_specs=pl.BlockSpec((_EB, L, CHARSET), lambda i: (i, 0, 0)),
        out_shape=jax.ShapeDtypeStruct((B, L, CHARSET), jnp.float32),
    )
    return compact(rows_p)


def kernel(index, targets, embedding_table):
    table3 = jnp.pad(
        embedding_table, ((0, 0), (0, _WPAD - CHARSET))).reshape(
            CHARSET, 8, 128)
    idx3 = index.astype(jnp.int32).reshape(_NW, _NCHUNK, _C)
    return _run(table3, idx3)


# speedup vs baseline: 1.4419x; 1.4419x over previous
import jax
import jax.numpy as jnp
from jax import lax
from jax.experimental import pallas as pl
from jax.experimental.pallas import tpu as pltpu
from jax.experimental.pallas import tpu_sc as plsc

_INFO = plsc.get_sparse_core_info()
_NC = _INFO.num_cores
_NS = _INFO.num_subcores
_NW = _NC * _NS

CHARSET = 1000
_WPAD = 1024
B, L = 4096, 20
_N = B * L
_PER_W = _N // _NW
_C = 20
_NCHUNK = _PER_W // _C
_NBUF = 4
_T = _NCHUNK // _NBUF
_REM = _NCHUNK - (_T - 1) * _NBUF

_BB = 256


def _gather_body(table_hbm, idx_hbm, out_hbm, idx_v,
                 r0, r1, r2, r3, g0, g1, g2, g3, w0, w1, w2, w3):
    bufs = [r0, r1, r2, r3]
    gs = [g0, g1, g2, g3]
    ws = [w0, w1, w2, w3]
    wid = lax.axis_index("s") * _NC + lax.axis_index("c")
    base = wid * _PER_W
    pltpu.sync_copy(idx_hbm.at[wid], idx_v)

    def start_gather(j, b):
        pltpu.async_copy(table_hbm.at[idx_v.at[j]], bufs[b], gs[b])

    def wait_gather(j, b):
        pltpu.make_async_copy(table_hbm.at[idx_v.at[j]], bufs[b], gs[b]).wait()

    def start_write(j, b):
        pltpu.async_copy(
            bufs[b], out_hbm.at[pl.ds(base + j * _C, _C)], ws[b])

    def wait_write(j, b):
        pltpu.make_async_copy(
            bufs[b], out_hbm.at[pl.ds(base + j * _C, _C)], ws[b]).wait()

    for b in range(_NBUF):
        start_gather(b, b)

    def outer(g, carry):
        jj = g * _NBUF
        for b in range(_NBUF):
            wait_gather(jj + b, b)
            start_write(jj + b, b)
        for b in range(_NBUF):
            wait_write(jj + b, b)
            start_gather(jj + _NBUF + b, b)
        return carry

    lax.fori_loop(0, _T - 1, outer, 0)

    jj = (_T - 1) * _NBUF
    for k in range(_REM):
        b = k % _NBUF
        wait_gather(jj + k, b)
        start_write(jj + k, b)
    for k in range(_REM):
        b = k % _NBUF
        wait_write(jj + k, b)


def _compact_body(x_ref, y_ref):
    x = x_ref[...].reshape(_BB, _WPAD)[:, :CHARSET]
    y_ref[...] = jnp.swapaxes(x, 0, 1)[None]


@jax.jit
def _run(table3, idx3):
    mesh = plsc.VectorSubcoreMesh(core_axis_name="c", subcore_axis_name="s")
    gather = pl.kernel(
        _gather_body,
        out_type=jax.ShapeDtypeStruct((_N, 8, 128), jnp.float32),
        mesh=mesh,
        scratch_types=(
            [pltpu.VMEM((_NCHUNK, _C), jnp.int32)]
            + [pltpu.VMEM((_C, 8, 128), jnp.float32) for _ in range(_NBUF)]
            + [pltpu.SemaphoreType.DMA for _ in range(2 * _NBUF)]
        ),
    )
    rows_p = gather(table3, idx3)
    compact = pl.pallas_call(
        _compact_body,
        grid=(L, B // _BB),
        in_specs=[pl.BlockSpec(
            (_BB, 8, 128), lambda l, i: (l * (B // _BB) + i, 0, 0))],
        out_specs=pl.BlockSpec((1, CHARSET, _BB), lambda l, i: (l, 0, i)),
        out_shape=jax.ShapeDtypeStruct((L, CHARSET, B), jnp.float32),
    )
    t = compact(rows_p)
    return jnp.transpose(t, (2, 0, 1))


def kernel(index, targets, embedding_table):
    table3 = jnp.pad(
        embedding_table, ((0, 0), (0, _WPAD - CHARSET))).reshape(
            CHARSET, 8, 128)
    idx3 = index.astype(jnp.int32).T.reshape(_NW, _NCHUNK, _C)
    return _run(table3, idx3)


# speedup vs baseline: 1.7012x; 1.1798x over previous
import jax
import jax.numpy as jnp
from jax import lax
from jax.experimental import pallas as pl
from jax.experimental.pallas import tpu as pltpu
from jax.experimental.pallas import tpu_sc as plsc

_INFO = plsc.get_sparse_core_info()
_NC = _INFO.num_cores
_NS = _INFO.num_subcores
_NW = _NC * _NS

CHARSET = 1000
_WPAD = 1024
B, L = 4096, 20
_N = B * L
_PER_W = _N // _NW
_C = 20
_NCHUNK = _PER_W // _C
_NBUF = 4
_T = _NCHUNK // _NBUF
_REM = _NCHUNK - (_T - 1) * _NBUF

_BB = 512


def _gather_body(table_hbm, idx_hbm, out_hbm, idx_v,
                 r0, r1, r2, r3, g0, g1, g2, g3, w0, w1, w2, w3):
    bufs = [r0, r1, r2, r3]
    gs = [g0, g1, g2, g3]
    ws = [w0, w1, w2, w3]
    wid = lax.axis_index("s") * _NC + lax.axis_index("c")
    base = wid * _PER_W
    pltpu.sync_copy(idx_hbm.at[wid], idx_v)

    def start_gather(j, b):
        pltpu.async_copy(table_hbm.at[idx_v.at[j]], bufs[b], gs[b])

    def wait_gather(j, b):
        pltpu.make_async_copy(table_hbm.at[idx_v.at[j]], bufs[b], gs[b]).wait()

    def start_write(j, b):
        pltpu.async_copy(
            bufs[b], out_hbm.at[pl.ds(base + j * _C, _C)], ws[b])

    def wait_write(j, b):
        pltpu.make_async_copy(
            bufs[b], out_hbm.at[pl.ds(base + j * _C, _C)], ws[b]).wait()

    for b in range(_NBUF):
        start_gather(b, b)

    def outer(g, carry):
        jj = g * _NBUF
        for b in range(_NBUF):
            wait_gather(jj + b, b)
            start_write(jj + b, b)
        for b in range(_NBUF):
            wait_write(jj + b, b)
            start_gather(jj + _NBUF + b, b)
        return carry

    lax.fori_loop(0, _T - 1, outer, 0)

    jj = (_T - 1) * _NBUF
    for k in range(_REM):
        b = k % _NBUF
        wait_gather(jj + k, b)
        start_write(jj + k, b)
    for k in range(_REM):
        b = k % _NBUF
        wait_write(jj + k, b)


def _compact_body(x_ref, y_ref):
    x = x_ref[...].reshape(_BB, _WPAD)[:, :CHARSET]
    y_ref[...] = jnp.swapaxes(x, 0, 1)[None]


@jax.jit
def _run(table3, idx3):
    mesh = plsc.VectorSubcoreMesh(core_axis_name="c", subcore_axis_name="s")
    gather = pl.kernel(
        _gather_body,
        out_type=jax.ShapeDtypeStruct((_N, 8, 128), jnp.float32),
        mesh=mesh,
        scratch_types=(
            [pltpu.VMEM((_NCHUNK, _C), jnp.int32)]
            + [pltpu.VMEM((_C, 8, 128), jnp.float32) for _ in range(_NBUF)]
            + [pltpu.SemaphoreType.DMA for _ in range(2 * _NBUF)]
        ),
    )
    rows_p = gather(table3, idx3)
    compact = pl.pallas_call(
        _compact_body,
        grid=(L, B // _BB),
        in_specs=[pl.BlockSpec(
            (_BB, 8, 128), lambda l, i: (l * (B // _BB) + i, 0, 0))],
        out_specs=pl.BlockSpec((1, CHARSET, _BB), lambda l, i: (l, 0, i)),
        out_shape=jax.ShapeDtypeStruct((L, CHARSET, B), jnp.float32),
    )
    t = compact(rows_p)
    return jnp.transpose(t, (2, 0, 1))


def kernel(index, targets, embedding_table):
    table3 = jnp.pad(
        embedding_table, ((0, 0), (0, _WPAD - CHARSET))).reshape(
            CHARSET, 8, 128)
    idx3 = index.astype(jnp.int32).T.reshape(_NW, _NCHUNK, _C)
    return _run(table3, idx3)


# speedup vs baseline: 1.8499x; 1.0874x over previous
import jax
import jax.numpy as jnp
from jax import lax
from jax.experimental import pallas as pl
from jax.experimental.pallas import tpu as pltpu
from jax.experimental.pallas import tpu_sc as plsc

_INFO = plsc.get_sparse_core_info()
_NC = _INFO.num_cores
_NS = _INFO.num_subcores
_NW = _NC * _NS

CHARSET = 1000
_WPAD = 1024
B, L = 4096, 20
_N = B * L
_PER_W = _N // _NW
_C = 20
_NCHUNK = _PER_W // _C
_NBUF = 4
_T = _NCHUNK // _NBUF
_REM = _NCHUNK - (_T - 1) * _NBUF

_BB = 1024


def _gather_body(table_hbm, idx_hbm, out_hbm, idx_v,
                 r0, r1, r2, r3, g0, g1, g2, g3, w0, w1, w2, w3):
    bufs = [r0, r1, r2, r3]
    gs = [g0, g1, g2, g3]
    ws = [w0, w1, w2, w3]
    wid = lax.axis_index("s") * _NC + lax.axis_index("c")
    base = wid * _PER_W
    pltpu.sync_copy(idx_hbm.at[wid], idx_v)

    def start_gather(j, b):
        pltpu.async_copy(table_hbm.at[idx_v.at[j]], bufs[b], gs[b])

    def wait_gather(j, b):
        pltpu.make_async_copy(table_hbm.at[idx_v.at[j]], bufs[b], gs[b]).wait()

    def start_write(j, b):
        pltpu.async_copy(
            bufs[b], out_hbm.at[pl.ds(base + j * _C, _C)], ws[b])

    def wait_write(j, b):
        pltpu.make_async_copy(
            bufs[b], out_hbm.at[pl.ds(base + j * _C, _C)], ws[b]).wait()

    for b in range(_NBUF):
        start_gather(b, b)

    def outer(g, carry):
        jj = g * _NBUF
        for b in range(_NBUF):
            wait_gather(jj + b, b)
            start_write(jj + b, b)
        for b in range(_NBUF):
            wait_write(jj + b, b)
            start_gather(jj + _NBUF + b, b)
        return carry

    lax.fori_loop(0, _T - 1, outer, 0)

    jj = (_T - 1) * _NBUF
    for k in range(_REM):
        b = k % _NBUF
        wait_gather(jj + k, b)
        start_write(jj + k, b)
    for k in range(_REM):
        b = k % _NBUF
        wait_write(jj + k, b)


def _compact_body(x_ref, y_ref):
    x = x_ref[...].reshape(_BB, _WPAD)[:, :CHARSET]
    y_ref[...] = jnp.swapaxes(x, 0, 1)[None]


@jax.jit
def _run(table3, idx3):
    mesh = plsc.VectorSubcoreMesh(core_axis_name="c", subcore_axis_name="s")
    gather = pl.kernel(
        _gather_body,
        out_type=jax.ShapeDtypeStruct((_N, 8, 128), jnp.float32),
        mesh=mesh,
        scratch_types=(
            [pltpu.VMEM((_NCHUNK, _C), jnp.int32)]
            + [pltpu.VMEM((_C, 8, 128), jnp.float32) for _ in range(_NBUF)]
            + [pltpu.SemaphoreType.DMA for _ in range(2 * _NBUF)]
        ),
    )
    rows_p = gather(table3, idx3)
    compact = pl.pallas_call(
        _compact_body,
        grid=(L, B // _BB),
        in_specs=[pl.BlockSpec(
            (_BB, 8, 128), lambda l, i: (l * (B // _BB) + i, 0, 0))],
        out_specs=pl.BlockSpec((1, CHARSET, _BB), lambda l, i: (l, 0, i)),
        out_shape=jax.ShapeDtypeStruct((L, CHARSET, B), jnp.float32),
    )
    t = compact(rows_p)
    return jnp.transpose(t, (2, 0, 1))


def kernel(index, targets, embedding_table):
    table3 = jnp.pad(
        embedding_table, ((0, 0), (0, _WPAD - CHARSET))).reshape(
            CHARSET, 8, 128)
    idx3 = index.astype(jnp.int32).T.reshape(_NW, _NCHUNK, _C)
    return _run(table3, idx3)


# speedup vs baseline: 1.8709x; 1.0113x over previous
import jax
import jax.numpy as jnp
from jax import lax
from jax.experimental import pallas as pl
from jax.experimental.pallas import tpu as pltpu
from jax.experimental.pallas import tpu_sc as plsc

_INFO = plsc.get_sparse_core_info()
_NC = _INFO.num_cores
_NS = _INFO.num_subcores
_NW = _NC * _NS

CHARSET = 1000
_WPAD = 1024
B, L = 4096, 20
_N = B * L
_PER_W = _N // _NW
_C = 20
_NCHUNK = _PER_W // _C
_NBUF = 4
_T = _NCHUNK // _NBUF
_REM = _NCHUNK - (_T - 1) * _NBUF

_BB = 2048


def _gather_body(table_hbm, idx_hbm, out_hbm, idx_v,
                 r0, r1, r2, r3, g0, g1, g2, g3, w0, w1, w2, w3):
    bufs = [r0, r1, r2, r3]
    gs = [g0, g1, g2, g3]
    ws = [w0, w1, w2, w3]
    wid = lax.axis_index("s") * _NC + lax.axis_index("c")
    base = wid * _PER_W
    pltpu.sync_copy(idx_hbm.at[wid], idx_v)

    def start_gather(j, b):
        pltpu.async_copy(table_hbm.at[idx_v.at[j]], bufs[b], gs[b])

    def wait_gather(j, b):
        pltpu.make_async_copy(table_hbm.at[idx_v.at[j]], bufs[b], gs[b]).wait()

    def start_write(j, b):
        pltpu.async_copy(
            bufs[b], out_hbm.at[pl.ds(base + j * _C, _C)], ws[b])

    def wait_write(j, b):
        pltpu.make_async_copy(
            bufs[b], out_hbm.at[pl.ds(base + j * _C, _C)], ws[b]).wait()

    for b in range(_NBUF):
        start_gather(b, b)

    def outer(g, carry):
        jj = g * _NBUF
        for b in range(_NBUF):
            wait_gather(jj + b, b)
            start_write(jj + b, b)
        for b in range(_NBUF):
            wait_write(jj + b, b)
            start_gather(jj + _NBUF + b, b)
        return carry

    lax.fori_loop(0, _T - 1, outer, 0)

    jj = (_T - 1) * _NBUF
    for k in range(_REM):
        b = k % _NBUF
        wait_gather(jj + k, b)
        start_write(jj + k, b)
    for k in range(_REM):
        b = k % _NBUF
        wait_write(jj + k, b)


def _compact_body(x_ref, y_ref):
    x = x_ref[...].reshape(_BB, _WPAD)[:, :CHARSET]
    y_ref[...] = jnp.swapaxes(x, 0, 1)[None]


@jax.jit
def _run(table3, idx3):
    mesh = plsc.VectorSubcoreMesh(core_axis_name="c", subcore_axis_name="s")
    gather = pl.kernel(
        _gather_body,
        out_type=jax.ShapeDtypeStruct((_N, 8, 128), jnp.float32),
        mesh=mesh,
        scratch_types=(
            [pltpu.VMEM((_NCHUNK, _C), jnp.int32)]
            + [pltpu.VMEM((_C, 8, 128), jnp.float32) for _ in range(_NBUF)]
            + [pltpu.SemaphoreType.DMA for _ in range(2 * _NBUF)]
        ),
    )
    rows_p = gather(table3, idx3)
    compact = pl.pallas_call(
        _compact_body,
        grid=(L, B // _BB),
        in_specs=[pl.BlockSpec(
            (_BB, 8, 128), lambda l, i: (l * (B // _BB) + i, 0, 0))],
        out_specs=pl.BlockSpec((1, CHARSET, _BB), lambda l, i: (l, 0, i)),
        out_shape=jax.ShapeDtypeStruct((L, CHARSET, B), jnp.float32),
    )
    t = compact(rows_p)
    return jnp.transpose(t, (2, 0, 1))


def kernel(index, targets, embedding_table):
    table3 = jnp.pad(
        embedding_table, ((0, 0), (0, _WPAD - CHARSET))).reshape(
            CHARSET, 8, 128)
    idx3 = index.astype(jnp.int32).T.reshape(_NW, _NCHUNK, _C)
    return _run(table3, idx3)


# speedup vs baseline: 1.8759x; 1.0027x over previous
import jax
import jax.numpy as jnp
from jax import lax
from jax.experimental import pallas as pl
from jax.experimental.pallas import tpu as pltpu
from jax.experimental.pallas import tpu_sc as plsc

_INFO = plsc.get_sparse_core_info()
_NC = _INFO.num_cores
_NS = _INFO.num_subcores
_NW = _NC * _NS

CHARSET = 1000
_WPAD = 1024
B, L = 4096, 20
_S = 4
_LS = L // _S
_NROWS = _LS * B
_PER_W = _NROWS // _NW
_C = 20
_NCHUNK = _PER_W // _C
_NBUF = 4
_T = _NCHUNK // _NBUF
_REM = _NCHUNK - (_T - 1) * _NBUF

_BB = 2048


def _gather_body(table_hbm, idx_hbm, out_hbm, idx_v,
                 r0, r1, r2, r3, g0, g1, g2, g3, w0, w1, w2, w3):
    bufs = [r0, r1, r2, r3]
    gs = [g0, g1, g2, g3]
    ws = [w0, w1, w2, w3]
    wid = lax.axis_index("s") * _NC + lax.axis_index("c")
    base = wid * _PER_W
    pltpu.sync_copy(idx_hbm.at[wid], idx_v)

    def start_gather(j, b):
        pltpu.async_copy(table_hbm.at[idx_v.at[j]], bufs[b], gs[b])

    def wait_gather(j, b):
        pltpu.make_async_copy(table_hbm.at[idx_v.at[j]], bufs[b], gs[b]).wait()

    def start_write(j, b):
        pltpu.async_copy(
            bufs[b], out_hbm.at[pl.ds(base + j * _C, _C)], ws[b])

    def wait_write(j, b):
        pltpu.make_async_copy(
            bufs[b], out_hbm.at[pl.ds(base + j * _C, _C)], ws[b]).wait()

    for b in range(_NBUF):
        start_gather(b, b)

    def outer(g, carry):
        jj = g * _NBUF
        for b in range(_NBUF):
            wait_gather(jj + b, b)
            start_write(jj + b, b)
        for b in range(_NBUF):
            wait_write(jj + b, b)
            start_gather(jj + _NBUF + b, b)
        return carry

    lax.fori_loop(0, _T - 1, outer, 0)

    jj = (_T - 1) * _NBUF
    for k in range(_REM):
        b = k % _NBUF
        wait_gather(jj + k, b)
        start_write(jj + k, b)
    for k in range(_REM):
        b = k % _NBUF
        wait_write(jj + k, b)


def _compact_first_body(x_ref, y_ref):
    x = x_ref[...].reshape(_BB, _WPAD)[:, :CHARSET]
    y_ref[...] = jnp.swapaxes(x, 0, 1)[None]


def _compact_next_body(x_ref, t_in_ref, y_ref):
    del t_in_ref
    x = x_ref[...].reshape(_BB, _WPAD)[:, :CHARSET]
    y_ref[...] = jnp.swapaxes(x, 0, 1)[None]


@jax.jit
def _run(table3, idx4):
    mesh = plsc.VectorSubcoreMesh(core_axis_name="c", subcore_axis_name="s")
    gather = pl.kernel(
        _gather_body,
        out_type=jax.ShapeDtypeStruct((_NROWS, 8, 128), jnp.float32),
        mesh=mesh,
        scratch_types=(
            [pltpu.VMEM((_NCHUNK, _C), jnp.int32)]
            + [pltpu.VMEM((_C, 8, 128), jnp.float32) for _ in range(_NBUF)]
            + [pltpu.SemaphoreType.DMA for _ in range(2 * _NBUF)]
        ),
    )
    rows = [gather(table3, idx4[s]) for s in range(_S)]

    t_shape = jax.ShapeDtypeStruct((L, CHARSET, B), jnp.float32)
    x_spec = pl.BlockSpec(
        (_BB, 8, 128), lambda l, i: (l * (B // _BB) + i, 0, 0))

    def y_spec(s):
        return pl.BlockSpec(
            (1, CHARSET, _BB), lambda l, i, s=s: (s * _LS + l, 0, i))

    t = pl.pallas_call(
        _compact_first_body,
        grid=(_LS, B // _BB),
        in_specs=[x_spec],
        out_specs=y_spec(0),
        out_shape=t_shape,
    )(rows[0])
    for s in range(1, _S):
        t = pl.pallas_call(
            _compact_next_body,
            grid=(_LS, B // _BB),
            in_specs=[x_spec, pl.BlockSpec(memory_space=pl.ANY)],
            out_specs=y_spec(s),
            out_shape=t_shape,
            input_output_aliases={1: 0},
        )(rows[s], t)
    return jnp.transpose(t, (2, 0, 1))


def kernel(index, targets, embedding_table):
    table3 = jnp.pad(
        embedding_table, ((0, 0), (0, _WPAD - CHARSET))).reshape(
            CHARSET, 8, 128)
    idx4 = index.astype(jnp.int32).T.reshape(_S, _NW, _NCHUNK, _C)
    return _run(table3, idx4)


# speedup vs baseline: 2.5344x; 1.3510x over previous
import jax
import jax.numpy as jnp
from jax import lax
from jax.experimental import pallas as pl
from jax.experimental.pallas import tpu as pltpu
from jax.experimental.pallas import tpu_sc as plsc

_INFO = plsc.get_sparse_core_info()
_NC = _INFO.num_cores
_NS = _INFO.num_subcores
_NW = _NC * _NS

CHARSET = 1000
_WPAD = 1024
B, L = 4096, 20
_S = 4
_LS = L // _S
_NROWS = _LS * B
_PER_W = _NROWS // _NW
_C = 10
_NCHUNK = _PER_W // _C
_NBUF = 4
_T = _NCHUNK // _NBUF
_REM = _NCHUNK - (_T - 1) * _NBUF

_BB = 2048


def _gather_body(table_hbm, idx_hbm, out_hbm, idx_v, table_sp,
                 r0, r1, r2, r3, g0, g1, g2, g3, w0, w1, w2, w3):
    bufs = [r0, r1, r2, r3]
    gs = [g0, g1, g2, g3]
    ws = [w0, w1, w2, w3]
    sid = lax.axis_index("s")
    wid = sid * _NC + lax.axis_index("c")
    base = wid * _PER_W
    @pl.when(sid == 0)
    def _load_table():
        pltpu.sync_copy(table_hbm, table_sp)

    plsc.subcore_barrier()
    pltpu.sync_copy(idx_hbm.at[wid], idx_v)

    def start_gather(j, b):
        pltpu.async_copy(table_sp.at[idx_v.at[j]], bufs[b], gs[b])

    def wait_gather(j, b):
        pltpu.make_async_copy(table_sp.at[idx_v.at[j]], bufs[b], gs[b]).wait()

    def start_write(j, b):
        pltpu.async_copy(
            bufs[b], out_hbm.at[pl.ds(base + j * _C, _C)], ws[b])

    def wait_write(j, b):
        pltpu.make_async_copy(
            bufs[b], out_hbm.at[pl.ds(base + j * _C, _C)], ws[b]).wait()

    for b in range(_NBUF):
        start_gather(b, b)

    def outer(g, carry):
        jj = g * _NBUF
        for b in range(_NBUF):
            wait_gather(jj + b, b)
            start_write(jj + b, b)
        for b in range(_NBUF):
            wait_write(jj + b, b)
            start_gather(jj + _NBUF + b, b)
        return carry

    lax.fori_loop(0, _T - 1, outer, 0)

    jj = (_T - 1) * _NBUF
    for k in range(_REM):
        b = k % _NBUF
        wait_gather(jj + k, b)
        start_write(jj + k, b)
    for k in range(_REM):
        b = k % _NBUF
        wait_write(jj + k, b)


def _compact_first_body(x_ref, y_ref):
    x = x_ref[...].reshape(_BB, _WPAD)[:, :CHARSET]
    y_ref[...] = jnp.swapaxes(x, 0, 1)[None]


def _compact_next_body(x_ref, t_in_ref, y_ref):
    del t_in_ref
    x = x_ref[...].reshape(_BB, _WPAD)[:, :CHARSET]
    y_ref[...] = jnp.swapaxes(x, 0, 1)[None]


@jax.jit
def _run(table3, idx4):
    mesh = plsc.VectorSubcoreMesh(core_axis_name="c", subcore_axis_name="s")
    gather = pl.kernel(
        _gather_body,
        out_type=jax.ShapeDtypeStruct((_NROWS, 8, 128), jnp.float32),
        mesh=mesh,
        scratch_types=(
            [pltpu.VMEM((_NCHUNK, _C), jnp.int32)]
            + [pltpu.VMEM_SHARED((CHARSET, 8, 128), jnp.float32)]
            + [pltpu.VMEM((_C, 8, 128), jnp.float32) for _ in range(_NBUF)]
            + [pltpu.SemaphoreType.DMA for _ in range(2 * _NBUF)]
        ),
    )
    rows = [gather(table3, idx4[s]) for s in range(_S)]

    t_shape = jax.ShapeDtypeStruct((L, CHARSET, B), jnp.float32)
    x_spec = pl.BlockSpec(
        (_BB, 8, 128), lambda l, i: (l * (B // _BB) + i, 0, 0))

    def y_spec(s):
        return pl.BlockSpec(
            (1, CHARSET, _BB), lambda l, i, s=s: (s * _LS + l, 0, i))

    t = pl.pallas_call(
        _compact_first_body,
        grid=(_LS, B // _BB),
        in_specs=[x_spec],
        out_specs=y_spec(0),
        out_shape=t_shape,
    )(rows[0])
    for s in range(1, _S):
        t = pl.pallas_call(
            _compact_next_body,
            grid=(_LS, B // _BB),
            in_specs=[x_spec, pl.BlockSpec(memory_space=pl.ANY)],
            out_specs=y_spec(s),
            out_shape=t_shape,
            input_output_aliases={1: 0},
        )(rows[s], t)
    return jnp.transpose(t, (2, 0, 1))


def kernel(index, targets, embedding_table):
    table3 = jnp.pad(
        embedding_table, ((0, 0), (0, _WPAD - CHARSET))).reshape(
            CHARSET, 8, 128)
    idx4 = index.astype(jnp.int32).T.reshape(_S, _NW, _NCHUNK, _C)
    return _run(table3, idx4)
